# Initial kernel scaffold; baseline (speedup 1.0000x reference)
import jax
import jax.numpy as jnp
from jax.experimental import pallas as pl


def kernel(input, target, mask):
    raise NotImplementedError("write your pallas kernel here")


# speedup vs baseline: 4.8436x; 4.8436x over previous
import math

import jax
import jax.numpy as jnp
from jax.experimental import pallas as pl
from jax.experimental.pallas import tpu as pltpu

_SMOOTHING = 0.1
_CONFIDENCE = 1.0 - _SMOOTHING


def _loss_kernel(inp_ref, tgt_ref, mask_ref, out_ref, acc_ref, *, nblocks, vb, V, C):
    b = pl.program_id(0)

    @pl.when(b == 0)
    def _init():
        acc_ref[:, :] = jnp.zeros_like(acc_ref)

    eps = _SMOOTHING / (V - 1)
    x = inp_ref[:, :]
    col = jax.lax.broadcasted_iota(jnp.int32, x.shape, 1) + b * vb
    x = jnp.where(col < V, x, 0.0)
    w = jnp.where(col == tgt_ref[:, :], _CONFIDENCE, eps)
    acc_ref[:, :] += jnp.sum(x * w, axis=1, keepdims=True)

    @pl.when(b == nblocks - 1)
    def _finish():
        m = mask_ref[:, :]
        num = jnp.sum(m * (C - acc_ref[:, :]), keepdims=True)
        den = jnp.sum(m, keepdims=True)
        out_ref[:, :] = num / den


def kernel(input, target, mask):
    S = input.shape[1]
    V = input.shape[-1]
    target = target[:, :S]
    mask = mask[:, :S]
    inp = input.reshape(-1, V)
    N = inp.shape[0]
    tgt = target.reshape(N, 1).astype(jnp.int32)
    m = mask.reshape(N, 1).astype(jnp.float32)

    eps = _SMOOTHING / (V - 1)
    C = (V - 1) * eps * math.log(eps) + _CONFIDENCE * math.log(_CONFIDENCE)

    VB = 2048
    nblocks = pl.cdiv(V, VB)

    import functools
    out = pl.pallas_call(
        functools.partial(_loss_kernel, nblocks=nblocks, vb=VB, V=V, C=C),
        grid=(nblocks,),
        in_specs=[
            pl.BlockSpec((N, VB), lambda b: (0, b)),
            pl.BlockSpec((N, 1), lambda b: (0, 0)),
            pl.BlockSpec((N, 1), lambda b: (0, 0)),
        ],
        out_specs=pl.BlockSpec((1, 1), lambda b: (0, 0)),
        out_shape=jax.ShapeDtypeStruct((1, 1), jnp.float32),
        scratch_shapes=[pltpu.VMEM((N, 1), jnp.float32)],
    )(inp, tgt, m)
    return out[0, 0]


# speedup vs baseline: 5.9594x; 1.2304x over previous
import math

import jax
import jax.numpy as jnp
from jax.experimental import pallas as pl
from jax.experimental.pallas import tpu as pltpu

_SMOOTHING = 0.1
_CONFIDENCE = 1.0 - _SMOOTHING


def _loss_kernel(inp_ref, tgt_ref, mask_ref, out_ref, acc_ref, *, nblocks, vb, V, C):
    b = pl.program_id(0)

    @pl.when(b == 0)
    def _init():
        acc_ref[:, :] = jnp.zeros_like(acc_ref)

    eps = _SMOOTHING / (V - 1)
    x = inp_ref[:, :]
    col = jax.lax.broadcasted_iota(jnp.int32, x.shape, 1) + b * vb
    x = jnp.where(col < V, x, 0.0)
    w = jnp.where(col == tgt_ref[:, :], _CONFIDENCE, eps)
    acc_ref[:, :] += jnp.sum(x * w, axis=1, keepdims=True)

    @pl.when(b == nblocks - 1)
    def _finish():
        m = mask_ref[:, :]
        num = jnp.sum(m * (C - acc_ref[:, :]), keepdims=True)
        den = jnp.sum(m, keepdims=True)
        out_ref[:, :] = num / den


def kernel(input, target, mask):
    S = input.shape[1]
    V = input.shape[-1]
    target = target[:, :S]
    mask = mask[:, :S]
    inp = input.reshape(-1, V)
    N = inp.shape[0]
    tgt = target.reshape(N, 1).astype(jnp.int32)
    m = mask.reshape(N, 1).astype(jnp.float32)

    eps = _SMOOTHING / (V - 1)
    C = (V - 1) * eps * math.log(eps) + _CONFIDENCE * math.log(_CONFIDENCE)

    VB = 8192
    nblocks = pl.cdiv(V, VB)

    import functools
    out = pl.pallas_call(
        functools.partial(_loss_kernel, nblocks=nblocks, vb=VB, V=V, C=C),
        grid=(nblocks,),
        in_specs=[
            pl.BlockSpec((N, VB), lambda b: (0, b)),
            pl.BlockSpec((N, 1), lambda b: (0, 0)),
            pl.BlockSpec((N, 1), lambda b: (0, 0)),
        ],
        out_specs=pl.BlockSpec((1, 1), lambda b: (0, 0)),
        out_shape=jax.ShapeDtypeStruct((1, 1), jnp.float32),
        scratch_shapes=[pltpu.VMEM((N, 1), jnp.float32)],
    )(inp, tgt, m)
    return out[0, 0]
